# baseline (speedup 1.0000x reference)
import functools

import jax
import jax.numpy as jnp
from jax import lax
from jax.experimental import pallas as pl
from jax.experimental.pallas import tpu as pltpu
from jax.experimental.pallas import tpu_sc as plsc


@functools.lru_cache(maxsize=None)
def _make_sc_gather(n_idx: int, emb_dim: int, vocab: int):
    info = plsc.get_sparse_core_info()
    nc, ns = info.num_cores, info.num_subcores
    nw = nc * ns
    assert n_idx % nw == 0
    b_per_w = n_idx // nw
    chunk = 80
    assert b_per_w % chunk == 0 and chunk % 8 == 0
    n_chunks = b_per_w // chunk

    mesh = plsc.VectorSubcoreMesh(core_axis_name="c", subcore_axis_name="s")

    @functools.partial(
        pl.kernel,
        mesh=mesh,
        compiler_params=pltpu.CompilerParams(use_tc_tiling_on_sc=False),
        out_type=jax.ShapeDtypeStruct((n_idx, emb_dim), jnp.float32),
        scratch_types=[
            pltpu.VMEM((n_chunks, chunk), jnp.int32),
            pltpu.VMEM((b_per_w, emb_dim), jnp.float32),
            pltpu.SemaphoreType.DMA,
        ],
    )
    def gather_kernel(emb_hbm, idx_hbm, out_hbm, idx_v, rows_v, sem):
        wid = lax.axis_index("s") * nc + lax.axis_index("c")
        base = wid * b_per_w
        pltpu.sync_copy(idx_hbm.at[wid], idx_v)
        copies = []
        for j in range(n_chunks):
            copies.append(
                pltpu.async_copy(
                    emb_hbm.at[idx_v.at[j]],
                    rows_v.at[pl.ds(j * chunk, chunk), :],
                    sem,
                )
            )
        for c in copies:
            c.wait()
        pltpu.sync_copy(rows_v, out_hbm.at[pl.ds(base, b_per_w)])

    def run(emb, idx):
        idx3 = idx.reshape(nw, n_chunks, chunk)
        return gather_kernel(emb, idx3)

    return run


def _mm_body(h_ref, w_ref, b_ref, o_ref):
    o_ref[...] = (
        jnp.dot(
            h_ref[...],
            w_ref[...],
            preferred_element_type=jnp.float32,
            precision=lax.Precision.HIGHEST,
        )
        + b_ref[...]
    )


@functools.lru_cache(maxsize=None)
def _make_tc_head(n_rows: int, emb_dim: int, vocab: int):
    bm = 512
    assert n_rows % bm == 0
    grid = (n_rows // bm,)
    return pl.pallas_call(
        _mm_body,
        grid=grid,
        in_specs=[
            pl.BlockSpec((bm, emb_dim), lambda i: (i, 0)),
            pl.BlockSpec((emb_dim, vocab), lambda i: (0, 0)),
            pl.BlockSpec((1, vocab), lambda i: (0, 0)),
        ],
        out_specs=pl.BlockSpec((bm, vocab), lambda i: (i, 0)),
        out_shape=jax.ShapeDtypeStruct((n_rows, vocab), jnp.float32),
    )


def kernel(x, emb, W, b):
    bsz, seq = x.shape
    vocab, emb_dim = emb.shape
    n_idx = bsz * seq
    idx = x.reshape(-1).astype(jnp.int32)
    h = _make_sc_gather(n_idx, emb_dim, vocab)(emb, idx)
    logits = _make_tc_head(n_idx, emb_dim, vocab)(h, W, b.reshape(1, vocab))
    return logits.reshape(bsz, seq, vocab)


# speedup vs baseline: 1.1565x; 1.1565x over previous
import functools

import jax
import jax.numpy as jnp
from jax import lax
from jax.experimental import pallas as pl
from jax.experimental.pallas import tpu as pltpu
from jax.experimental.pallas import tpu_sc as plsc

_KPAD = 128


@functools.lru_cache(maxsize=None)
def _make_sc_gather(n_idx: int):
    info = plsc.get_sparse_core_info()
    nc, ns = info.num_cores, info.num_subcores
    nw = nc * ns
    assert n_idx % nw == 0
    b_per_w = n_idx // nw
    chunk = 80
    assert b_per_w % chunk == 0 and chunk % 8 == 0
    n_chunks = b_per_w // chunk

    mesh = plsc.VectorSubcoreMesh(core_axis_name="c", subcore_axis_name="s")

    @functools.partial(
        pl.kernel,
        mesh=mesh,
        out_type=jax.ShapeDtypeStruct((n_idx, _KPAD), jnp.float32),
        scratch_types=[
            pltpu.VMEM((n_chunks, chunk), jnp.int32),
            pltpu.VMEM((chunk, _KPAD), jnp.float32),
            pltpu.VMEM((chunk, _KPAD), jnp.float32),
            pltpu.SemaphoreType.DMA,
            pltpu.SemaphoreType.DMA,
        ],
    )
    def gather_kernel(emb_hbm, idx_hbm, out_hbm, idx_v, buf0, buf1, sem0, sem1):
        wid = lax.axis_index("s") * nc + lax.axis_index("c")
        base = wid * b_per_w
        pltpu.sync_copy(idx_hbm.at[wid], idx_v)
        bufs = (buf0, buf1)
        sems = (sem0, sem1)
        gathers = [None] * n_chunks
        gathers[0] = pltpu.async_copy(emb_hbm.at[idx_v.at[0]], bufs[0], sems[0])
        for j in range(n_chunks):
            if j + 1 < n_chunks:
                gathers[j + 1] = pltpu.async_copy(
                    emb_hbm.at[idx_v.at[j + 1]], bufs[(j + 1) % 2], sems[(j + 1) % 2]
                )
            gathers[j].wait()
            pltpu.sync_copy(bufs[j % 2], out_hbm.at[pl.ds(base + j * chunk, chunk)])

    def run(emb_pad, idx):
        idx3 = idx.reshape(nw, n_chunks, chunk)
        return gather_kernel(emb_pad, idx3)

    return run


def _mm_body(h_ref, w_ref, b_ref, o_ref):
    o_ref[...] = (
        jnp.dot(h_ref[...], w_ref[...], preferred_element_type=jnp.float32)
        + b_ref[...]
    )


@functools.lru_cache(maxsize=None)
def _make_tc_head(n_rows: int, vocab: int):
    bm = 512
    assert n_rows % bm == 0
    grid = (n_rows // bm,)
    return pl.pallas_call(
        _mm_body,
        grid=grid,
        in_specs=[
            pl.BlockSpec((bm, _KPAD), lambda i: (i, 0)),
            pl.BlockSpec((_KPAD, vocab), lambda i: (0, 0)),
            pl.BlockSpec((1, vocab), lambda i: (0, 0)),
        ],
        out_specs=pl.BlockSpec((bm, vocab), lambda i: (i, 0)),
        out_shape=jax.ShapeDtypeStruct((n_rows, vocab), jnp.float32),
    )


def kernel(x, emb, W, b):
    bsz, seq = x.shape
    vocab, emb_dim = emb.shape
    n_idx = bsz * seq
    idx = x.reshape(-1).astype(jnp.int32)
    emb_pad = jnp.pad(emb, ((0, 0), (0, _KPAD - emb_dim)))
    w_pad = jnp.pad(W, ((0, _KPAD - emb_dim), (0, 0)))
    h = _make_sc_gather(n_idx)(emb_pad, idx)
    logits = _make_tc_head(n_idx, vocab)(h, w_pad, b.reshape(1, vocab))
    return logits.reshape(bsz, seq, vocab)


# speedup vs baseline: 3.3205x; 2.8711x over previous
import functools

import jax
import jax.numpy as jnp
from jax import lax
from jax.experimental import pallas as pl
from jax.experimental.pallas import tpu as pltpu
from jax.experimental.pallas import tpu_sc as plsc

_KPAD = 128


@functools.lru_cache(maxsize=None)
def _make_sc_gather(n_idx: int):
    info = plsc.get_sparse_core_info()
    nc, ns = info.num_cores, info.num_subcores
    nw = nc * ns
    assert n_idx % nw == 0
    b_per_w = n_idx // nw
    chunk = 80
    assert b_per_w % chunk == 0 and chunk % 8 == 0
    n_chunks = b_per_w // chunk

    mesh = plsc.VectorSubcoreMesh(core_axis_name="c", subcore_axis_name="s")

    @functools.partial(
        pl.kernel,
        mesh=mesh,
        out_type=jax.ShapeDtypeStruct((n_idx, _KPAD), jnp.float32),
        scratch_types=[
            pltpu.VMEM((n_chunks, chunk), jnp.int32),
            pltpu.VMEM((chunk, _KPAD), jnp.float32),
            pltpu.VMEM((chunk, _KPAD), jnp.float32),
            pltpu.SemaphoreType.DMA,
            pltpu.SemaphoreType.DMA,
        ],
    )
    def gather_kernel(emb_hbm, idx_hbm, out_hbm, idx_v, buf0, buf1, sem0, sem1):
        wid = lax.axis_index("s") * nc + lax.axis_index("c")
        base = wid * b_per_w
        pltpu.sync_copy(idx_hbm.at[wid], idx_v)
        bufs = (buf0, buf1)
        sems = (sem0, sem1)
        gathers = [None] * n_chunks
        gathers[0] = pltpu.async_copy(emb_hbm.at[idx_v.at[0]], bufs[0], sems[0])
        for j in range(n_chunks):
            if j + 1 < n_chunks:
                gathers[j + 1] = pltpu.async_copy(
                    emb_hbm.at[idx_v.at[j + 1]], bufs[(j + 1) % 2], sems[(j + 1) % 2]
                )
            gathers[j].wait()
            pltpu.sync_copy(bufs[j % 2], out_hbm.at[pl.ds(base + j * chunk, chunk)])

    def run(emb_pad, idx):
        idx3 = idx.reshape(nw, n_chunks, chunk)
        return gather_kernel(emb_pad, idx3)

    return run


def _mm_body(h_ref, wt_ref, b_ref, o_ref):
    prod = lax.dot_general(
        wt_ref[...],
        h_ref[0],
        (((1,), (1,)), ((), ())),
        preferred_element_type=jnp.float32,
    )
    o_ref[0] = prod + b_ref[...]


@functools.lru_cache(maxsize=None)
def _make_tc_head(seq: int, bsz: int, vocab: int):
    bm = 512
    assert bsz % bm == 0
    nt = bsz // bm
    grid = (seq, nt)
    return pl.pallas_call(
        _mm_body,
        grid=grid,
        in_specs=[
            pl.BlockSpec((1, bm, _KPAD), lambda l, t: (l, t, 0)),
            pl.BlockSpec((vocab, _KPAD), lambda l, t: (0, 0)),
            pl.BlockSpec((vocab, 1), lambda l, t: (0, 0)),
        ],
        out_specs=pl.BlockSpec((1, vocab, bm), lambda l, t: (l, 0, t)),
        out_shape=jax.ShapeDtypeStruct((seq, vocab, bsz), jnp.float32),
    )


def kernel(x, emb, W, b):
    bsz, seq = x.shape
    vocab, emb_dim = emb.shape
    n_idx = bsz * seq
    idx = jnp.swapaxes(x, 0, 1).reshape(-1).astype(jnp.int32)
    emb_pad = jnp.pad(emb, ((0, 0), (0, _KPAD - emb_dim)))
    w_t = jnp.pad(W, ((0, _KPAD - emb_dim), (0, 0))).T
    h = _make_sc_gather(n_idx)(emb_pad, idx)
    h3 = h.reshape(seq, bsz, _KPAD)
    out_t = _make_tc_head(seq, bsz, vocab)(h3, w_t, b.reshape(vocab, 1))
    return jnp.transpose(out_t, (2, 0, 1))


# speedup vs baseline: 4.0362x; 1.2155x over previous
import functools

import jax
import jax.numpy as jnp
from jax import lax
from jax.experimental import pallas as pl
from jax.experimental.pallas import tpu as pltpu
from jax.experimental.pallas import tpu_sc as plsc

_KPAD = 128


@functools.lru_cache(maxsize=None)
def _make_sc_gather(n_idx: int):
    info = plsc.get_sparse_core_info()
    nc, ns = info.num_cores, info.num_subcores
    nw = nc * ns
    assert n_idx % nw == 0
    b_per_w = n_idx // nw
    chunk = 80
    assert b_per_w % chunk == 0 and chunk % 8 == 0
    n_chunks = b_per_w // chunk

    mesh = plsc.VectorSubcoreMesh(core_axis_name="c", subcore_axis_name="s")

    @functools.partial(
        pl.kernel,
        mesh=mesh,
        out_type=jax.ShapeDtypeStruct((n_idx, _KPAD), jnp.float32),
        scratch_types=[
            pltpu.VMEM((n_chunks, chunk), jnp.int32),
            pltpu.VMEM((chunk, _KPAD), jnp.float32),
            pltpu.VMEM((chunk, _KPAD), jnp.float32),
            pltpu.SemaphoreType.DMA,
            pltpu.SemaphoreType.DMA,
        ],
    )
    def gather_kernel(emb_hbm, idx_hbm, out_hbm, idx_v, buf0, buf1, sem0, sem1):
        wid = lax.axis_index("s") * nc + lax.axis_index("c")
        base = wid * b_per_w
        pltpu.sync_copy(idx_hbm.at[wid], idx_v)
        bufs = (buf0, buf1)
        sems = (sem0, sem1)
        gathers = [None] * n_chunks
        gathers[0] = pltpu.async_copy(emb_hbm.at[idx_v.at[0]], bufs[0], sems[0])
        for j in range(n_chunks):
            if j + 1 < n_chunks:
                gathers[j + 1] = pltpu.async_copy(
                    emb_hbm.at[idx_v.at[j + 1]], bufs[(j + 1) % 2], sems[(j + 1) % 2]
                )
            gathers[j].wait()
            pltpu.sync_copy(bufs[j % 2], out_hbm.at[pl.ds(base + j * chunk, chunk)])

    def run(emb_pad, idx):
        idx3 = idx.reshape(nw, n_chunks, chunk)
        return gather_kernel(emb_pad, idx3)

    return run


def _mm_body(h_ref, wt_ref, b_ref, o_ref):
    prod = lax.dot_general(
        wt_ref[...],
        h_ref[0],
        (((1,), (1,)), ((), ())),
        preferred_element_type=jnp.float32,
    )
    o_ref[0] = prod + b_ref[...]


@functools.lru_cache(maxsize=None)
def _make_tc_head(seq: int, bsz: int, vocab: int):
    bm = 1024
    assert bsz % bm == 0
    nt = bsz // bm
    grid = (seq, nt)
    return pl.pallas_call(
        _mm_body,
        grid=grid,
        in_specs=[
            pl.BlockSpec((1, bm, _KPAD), lambda l, t: (l, t, 0)),
            pl.BlockSpec((vocab, _KPAD), lambda l, t: (0, 0)),
            pl.BlockSpec((vocab, 1), lambda l, t: (0, 0)),
        ],
        out_specs=pl.BlockSpec((1, vocab, bm), lambda l, t: (l, 0, t)),
        out_shape=jax.ShapeDtypeStruct((seq, vocab, bsz), jnp.float32),
    )


def kernel(x, emb, W, b):
    bsz, seq = x.shape
    vocab, emb_dim = emb.shape
    n_idx = bsz * seq
    idx = jnp.swapaxes(x, 0, 1).reshape(-1).astype(jnp.int32)
    emb_pad = jnp.pad(emb, ((0, 0), (0, _KPAD - emb_dim)))
    w_t = jnp.pad(W, ((0, _KPAD - emb_dim), (0, 0))).T
    h = _make_sc_gather(n_idx)(emb_pad, idx)
    h3 = h.reshape(seq, bsz, _KPAD)
    out_t = _make_tc_head(seq, bsz, vocab)(h3, w_t, b.reshape(vocab, 1))
    return jnp.transpose(out_t, (2, 0, 1))


# speedup vs baseline: 4.0803x; 1.0109x over previous
import functools

import jax
import jax.numpy as jnp
from jax import lax
from jax.experimental import pallas as pl
from jax.experimental.pallas import tpu as pltpu
from jax.experimental.pallas import tpu_sc as plsc

_KPAD = 128


@functools.lru_cache(maxsize=None)
def _make_sc_gather(n_idx: int):
    info = plsc.get_sparse_core_info()
    nc, ns = info.num_cores, info.num_subcores
    nw = nc * ns
    assert n_idx % nw == 0
    b_per_w = n_idx // nw
    chunk = 80
    assert b_per_w % chunk == 0 and chunk % 8 == 0
    n_chunks = b_per_w // chunk

    mesh = plsc.VectorSubcoreMesh(core_axis_name="c", subcore_axis_name="s")

    nb = 8

    @functools.partial(
        pl.kernel,
        mesh=mesh,
        compiler_params=pltpu.CompilerParams(use_tc_tiling_on_sc=True),
        out_type=jax.ShapeDtypeStruct((n_idx, _KPAD), jnp.float32),
        scratch_types=[
            pltpu.VMEM((n_chunks, chunk), jnp.int32),
        ]
        + [pltpu.VMEM((chunk, _KPAD), jnp.float32) for _ in range(nb)]
        + [pltpu.SemaphoreType.DMA for _ in range(2 * nb)],
    )
    def gather_kernel(emb_hbm, idx_hbm, out_hbm, idx_v, *scratch):
        bufs = scratch[:nb]
        gsems = scratch[nb : 2 * nb]
        osems = scratch[2 * nb : 3 * nb]
        wid = lax.axis_index("s") * nc + lax.axis_index("c")
        base = wid * b_per_w
        pltpu.sync_copy(idx_hbm.at[wid], idx_v)
        gathers = [None] * n_chunks
        outs = [None] * n_chunks
        for k in range(min(nb - 1, n_chunks)):
            gathers[k] = pltpu.async_copy(
                emb_hbm.at[idx_v.at[k]], bufs[k % nb], gsems[k % nb]
            )
        for j in range(n_chunks):
            k = j + nb - 1
            if k < n_chunks:
                if j >= 1:
                    outs[j - 1].wait()
                gathers[k] = pltpu.async_copy(
                    emb_hbm.at[idx_v.at[k]], bufs[k % nb], gsems[k % nb]
                )
            gathers[j].wait()
            outs[j] = pltpu.async_copy(
                bufs[j % nb],
                out_hbm.at[pl.ds(base + j * chunk, chunk)],
                osems[j % nb],
            )
        for j in range(max(0, n_chunks - nb), n_chunks):
            if outs[j] is not None:
                outs[j].wait()

    def run(emb_pad, idx):
        idx3 = idx.reshape(nw, n_chunks, chunk)
        return gather_kernel(emb_pad, idx3)

    return run


def _mm_body(h_ref, wt_ref, b_ref, o_ref):
    prod = lax.dot_general(
        wt_ref[...],
        h_ref[0],
        (((1,), (1,)), ((), ())),
        preferred_element_type=jnp.float32,
    )
    o_ref[0] = prod + b_ref[...]


@functools.lru_cache(maxsize=None)
def _make_tc_head(seq: int, bsz: int, vocab: int):
    bm = 1024
    assert bsz % bm == 0
    nt = bsz // bm
    grid = (seq, nt)
    return pl.pallas_call(
        _mm_body,
        grid=grid,
        in_specs=[
            pl.BlockSpec((1, bm, _KPAD), lambda l, t: (l, t, 0)),
            pl.BlockSpec((vocab, _KPAD), lambda l, t: (0, 0)),
            pl.BlockSpec((vocab, 1), lambda l, t: (0, 0)),
        ],
        out_specs=pl.BlockSpec((1, vocab, bm), lambda l, t: (l, 0, t)),
        out_shape=jax.ShapeDtypeStruct((seq, vocab, bsz), jnp.float32),
    )


def kernel(x, emb, W, b):
    bsz, seq = x.shape
    vocab, emb_dim = emb.shape
    n_idx = bsz * seq
    idx = jnp.swapaxes(x, 0, 1).reshape(-1).astype(jnp.int32)
    emb_pad = jnp.pad(emb, ((0, 0), (0, _KPAD - emb_dim)))
    w_t = jnp.pad(W, ((0, _KPAD - emb_dim), (0, 0))).T
    h = _make_sc_gather(n_idx)(emb_pad, idx)
    h3 = h.reshape(seq, bsz, _KPAD)
    out_t = _make_tc_head(seq, bsz, vocab)(h3, w_t, b.reshape(vocab, 1))
    return jnp.transpose(out_t, (2, 0, 1))
